# baseline (speedup 1.0000x reference)
import functools

import jax
import jax.numpy as jnp
from jax import lax
from jax.experimental import pallas as pl
from jax.experimental.pallas import tpu as pltpu
from jax.experimental.pallas import tpu_sc as plsc

_N = 10000
_E = 160000
_C = 128
_NB = 8
_G = 100
_R_MAX = 5.0
_P = 5
_AVG_N = 16.0

_NPAD = 10240
_GPAD = 128

_NC = 2
_NS = 16
_NW = _NC * _NS
_K = 128
_EP = 163840
_EPW = _EP // _NW
_NCHUNK = _EPW // _K
_NROWS_SUB = _NPAD // _NS

_sc_mesh = plsc.VectorSubcoreMesh(core_axis_name="c", subcore_axis_name="s")


def _edge_compute(k_edges, cw, fn):
    def rows(i, _):
        for rr in range(4):
            r = i * 4 + rr
            for j in range(cw // 16):
                fn(r, pl.ds(j * 16, 16))
        return 0

    lax.fori_loop(0, k_edges // 4, rows, 0, unroll=False)


def _make_edge_pass(has_ga, has_r0, scatter_sel, has_write, k_edges, cw, mslots=4):
    nchunk = _EP // _NW // k_edges
    outs = []
    if has_write:
        outs.append(jax.ShapeDtypeStruct((_EP, cw), jnp.float32))
    if scatter_sel is not None:
        outs.append(jax.ShapeDtypeStruct((_NC * _NPAD, cw), jnp.float32))
    scratch = []
    scratch.append(pltpu.VMEM((4, k_edges), jnp.int32))
    scratch.append(pltpu.SemaphoreType.DMA((4,)))
    if has_ga:
        scratch.append(pltpu.VMEM((4, k_edges), jnp.int32))
        scratch.append(pltpu.SemaphoreType.DMA((4,)))
    if scatter_sel is not None:
        scratch.append(pltpu.VMEM((mslots, k_edges), jnp.int32))
        scratch.append(pltpu.SemaphoreType.DMA((mslots,)))
    hslots = 4 if has_write else 2
    scratch.append(pltpu.VMEM((hslots, k_edges, cw), jnp.float32))
    scratch.append(pltpu.SemaphoreType.DMA((hslots,)))
    if has_ga:
        scratch.append(pltpu.VMEM((2, k_edges, cw), jnp.float32))
        scratch.append(pltpu.SemaphoreType.DMA((2,)))
    if has_r0:
        scratch.append(pltpu.VMEM((mslots, k_edges, cw), jnp.float32))
        scratch.append(pltpu.SemaphoreType.DMA((mslots,)))
    if has_write:
        scratch.append(pltpu.SemaphoreType.DMA((4,)))
    if scatter_sel is not None:
        scratch.append(pltpu.VMEM_SHARED((_NPAD, cw), jnp.float32))
        scratch.append(pltpu.SemaphoreType.DMA((mslots,)))

    @functools.partial(pl.kernel, out_type=tuple(outs) if len(outs) > 1
                       else outs[0], mesh=_sc_mesh, scratch_types=scratch,
                       compiler_params=pltpu.CompilerParams(
                           needs_layout_passes=False))
    def k(*refs):
        it = iter(refs)
        h_hbm = next(it)
        ga_hbm = next(it) if has_ga else None
        r0_hbm = next(it) if has_r0 else None
        send_hbm = next(it)
        recv_hbm = next(it)
        z_hbm = next(it) if scatter_sel is not None else None
        gr_hbm = next(it) if has_write else None
        out_hbm = next(it) if scatter_sel is not None else None
        hidx = next(it); sem_hidx = next(it)
        if has_ga:
            gaidx = next(it); sem_gaidx = next(it)
        if scatter_sel is not None:
            scidx = next(it); sem_scidx = next(it)
        hbuf = next(it); sem_h = next(it)
        if has_ga:
            gabuf = next(it); sem_ga = next(it)
        if has_r0:
            mbuf = next(it); sem_m = next(it)
        if has_write:
            sem_w = next(it)
        if scatter_sel is not None:
            acc = next(it); sem_sc = next(it)

        cid = lax.axis_index("c")
        sid = lax.axis_index("s")
        base = (cid * _NS + sid) * (_EP // _NW)
        sc_hbm = send_hbm if scatter_sel == "send" else recv_hbm

        if scatter_sel is not None:
            pltpu.sync_copy(z_hbm,
                            acc.at[pl.ds(sid * _NROWS_SUB, _NROWS_SUB)])
            plsc.subcore_barrier()

        def start_gidx(c, slot):
            off = base + c * k_edges
            pltpu.async_copy(send_hbm.at[pl.ds(off, k_edges)],
                             hidx.at[slot], sem_hidx.at[slot])
            if has_ga:
                pltpu.async_copy(recv_hbm.at[pl.ds(off, k_edges)],
                                 gaidx.at[slot], sem_gaidx.at[slot])

        def wait_gidx(slot):
            pltpu.make_async_copy(send_hbm.at[pl.ds(0, k_edges)],
                                  hidx.at[slot], sem_hidx.at[slot]).wait()
            if has_ga:
                pltpu.make_async_copy(recv_hbm.at[pl.ds(0, k_edges)],
                                      gaidx.at[slot], sem_gaidx.at[slot]).wait()

        def start_scidx(c, slot):
            off = base + c * k_edges
            pltpu.async_copy(sc_hbm.at[pl.ds(off, k_edges)],
                             scidx.at[slot], sem_scidx.at[slot])

        def start_rows(c, slot2, slot4, slotm):
            off = base + c * k_edges
            hs = slot4 if has_write else slot2
            pltpu.async_copy(h_hbm.at[hidx.at[slot4]], hbuf.at[hs],
                             sem_h.at[hs])
            if has_ga:
                pltpu.async_copy(ga_hbm.at[gaidx.at[slot4]], gabuf.at[slot2],
                                 sem_ga.at[slot2])
            if has_r0:
                pltpu.async_copy(r0_hbm.at[pl.ds(off, k_edges)],
                                 mbuf.at[slotm], sem_m.at[slotm])

        def wait_rows(slot2, slot4, slotm):
            hs = slot4 if has_write else slot2
            pltpu.make_async_copy(h_hbm.at[hidx.at[slot4]], hbuf.at[hs],
                                  sem_h.at[hs]).wait()
            if has_ga:
                pltpu.make_async_copy(ga_hbm.at[gaidx.at[slot4]],
                                      gabuf.at[slot2], sem_ga.at[slot2]).wait()
            if has_r0:
                pltpu.make_async_copy(r0_hbm.at[pl.ds(0, k_edges)],
                                      mbuf.at[slotm], sem_m.at[slotm]).wait()

        def wait_writes(slot4, slotm):
            if has_write:
                pltpu.make_async_copy(hbuf.at[slot4],
                                      gr_hbm.at[pl.ds(0, k_edges)],
                                      sem_w.at[slot4]).wait()
            if scatter_sel is not None:
                pltpu.make_async_copy(mbuf.at[slotm] if has_r0
                                      else hbuf.at[slot4],
                                      acc.at[scidx.at[slotm]],
                                      sem_sc.at[slotm]).wait()

        def compute(slot2, slot4, slotm):
            hs = slot4 if has_write else slot2
            if has_ga and has_r0:
                def fn(r, sl):
                    va = gabuf[slot2, r, sl]
                    hbuf[hs, r, sl] = va * hbuf[hs, r, sl]
                    mbuf[slotm, r, sl] = va * mbuf[slotm, r, sl]
            elif has_ga:
                def fn(r, sl):
                    hbuf[hs, r, sl] = gabuf[slot2, r, sl] * hbuf[hs, r, sl]
            else:
                def fn(r, sl):
                    mbuf[slotm, r, sl] = mbuf[slotm, r, sl] * hbuf[hs, r, sl]
            _edge_compute(k_edges, cw, fn)

        def start_writes(c, slot4, slotm):
            off = base + c * k_edges
            if has_write:
                pltpu.async_copy(hbuf.at[slot4], gr_hbm.at[pl.ds(off, k_edges)],
                                 sem_w.at[slot4])
            if scatter_sel is not None:
                pltpu.async_copy(mbuf.at[slotm] if has_r0 else hbuf.at[slot4],
                                 acc.at[scidx.at[slotm]], sem_sc.at[slotm],
                                 add=True)

        start_gidx(0, 0)
        start_gidx(1, 1)
        if scatter_sel is not None:
            start_scidx(0, 0)
        wait_gidx(0)
        start_rows(0, 0, 0, 0)

        def body(i, _):
            s2 = lax.rem(i, 2)
            s4 = lax.rem(i, 4)
            sm = lax.rem(i, mslots)
            n2 = lax.rem(i + 1, 2)
            n4 = lax.rem(i + 1, 4)
            nm = lax.rem(i + 1, mslots)

            @pl.when(jnp.logical_and(i + 1 < nchunk, i >= mslots - 1))
            def _():
                wait_writes(n4, nm)

            @pl.when(i + 1 < nchunk)
            def _():
                wait_gidx(n4)
                start_rows(i + 1, n2, n4, nm)

            @pl.when(i + 2 < nchunk)
            def _():
                start_gidx(i + 2, lax.rem(i + 2, 4))

            if scatter_sel is not None:
                @pl.when(i + 1 < nchunk)
                def _():
                    start_scidx(i + 1, nm)

            wait_rows(s2, s4, sm)
            compute(s2, s4, sm)
            if scatter_sel is not None:
                pltpu.make_async_copy(sc_hbm.at[pl.ds(0, k_edges)],
                                      scidx.at[sm], sem_scidx.at[sm]).wait()
            start_writes(i, s4, sm)
            return 0

        lax.fori_loop(0, nchunk, body, 0, unroll=False)
        for kk_ in range(max(0, nchunk - mslots), nchunk):
            wait_writes(kk_ % 4, kk_ % mslots)

        if scatter_sel is not None:
            plsc.subcore_barrier()
            pltpu.sync_copy(acc.at[pl.ds(sid * _NROWS_SUB, _NROWS_SUB)],
                            out_hbm.at[pl.ds(cid * _NPAD + sid * _NROWS_SUB,
                                             _NROWS_SUB)])

    return k


_agg0_pass = _make_edge_pass(False, True, "recv", False, 64, 128, mslots=3)
_bwd1_pass = _make_edge_pass(True, True, "send", True, 32, 128)
_bwd0_pass = _make_edge_pass(True, False, None, True, 128, 128)


def _agg0_full(h_tab, r0, send_p, recv_p):
    zc = jnp.zeros((_NROWS_SUB, _C), jnp.float32)
    p = _agg0_pass(h_tab, r0, send_p, recv_p, zc).reshape(_NC, _NPAD, _C)
    return p[0, :_N] + p[1, :_N]

_KV = 128
_NCHV = _EP // _NW // _KV


_KB = 64
_NCHB = _EP // _NW // _KB


@functools.partial(
    pl.kernel,
    out_type=jax.ShapeDtypeStruct((_NW * _NPAD * 4,), jnp.float32),
    mesh=_sc_mesh,
    compiler_params=pltpu.CompilerParams(needs_layout_passes=False),
    scratch_types=[
        pltpu.VMEM((4, _KB), jnp.int32),
        pltpu.VMEM((4, _KB), jnp.int32),
        pltpu.VMEM((2, _KB, _C), jnp.float32),
        pltpu.VMEM((2, _KB, _C), jnp.float32),
        pltpu.VMEM((2, _KB, _C), jnp.float32),
        pltpu.VMEM((2, 4, _KB), jnp.float32),
        pltpu.VMEM((_NPAD * 4,), jnp.float32),
        pltpu.SemaphoreType.DMA((4,)),
        pltpu.SemaphoreType.DMA((4,)),
        pltpu.SemaphoreType.DMA((2,)),
        pltpu.SemaphoreType.DMA((2,)),
        pltpu.SemaphoreType.DMA((2,)),
        pltpu.SemaphoreType.DMA((2,)),
        pltpu.SemaphoreType.DMA,
    ],
)
def _sc_dipole_pass(h_hbm, gw_hbm, r1_hbm, u4_hbm, send_hbm, recv_hbm,
                    z4_hbm, out_hbm, sidx, ridx, hbuf, rbuf, gbuf, ubuf,
                    facc, sem_si, sem_ri, sem_h, sem_r1, sem_gw, sem_u, sem0):
    cid = lax.axis_index("c")
    sid = lax.axis_index("s")
    w = cid * _NS + sid
    base = w * (_EP // _NW)

    pltpu.async_copy(z4_hbm, facc, sem0).wait()

    def start_idx(c, slot):
        off = base + c * _KB
        pltpu.async_copy(send_hbm.at[pl.ds(off, _KB)], sidx.at[slot],
                         sem_si.at[slot])
        pltpu.async_copy(recv_hbm.at[pl.ds(off, _KB)], ridx.at[slot],
                         sem_ri.at[slot])

    def wait_idx(slot):
        pltpu.make_async_copy(send_hbm.at[pl.ds(0, _KB)], sidx.at[slot],
                              sem_si.at[slot]).wait()
        pltpu.make_async_copy(recv_hbm.at[pl.ds(0, _KB)], ridx.at[slot],
                              sem_ri.at[slot]).wait()

    def start_rows(c, slot2, slot4):
        off = base + c * _KB
        pltpu.async_copy(h_hbm.at[sidx.at[slot4]], hbuf.at[slot2],
                         sem_h.at[slot2])
        pltpu.async_copy(gw_hbm.at[ridx.at[slot4]], gbuf.at[slot2],
                         sem_gw.at[slot2])
        pltpu.async_copy(r1_hbm.at[pl.ds(off, _KB)], rbuf.at[slot2],
                         sem_r1.at[slot2])
        pltpu.async_copy(u4_hbm.at[base // _KB + c], ubuf.at[slot2],
                         sem_u.at[slot2])

    def wait_rows(slot2, slot4):
        pltpu.make_async_copy(h_hbm.at[sidx.at[slot4]], hbuf.at[slot2],
                              sem_h.at[slot2]).wait()
        pltpu.make_async_copy(gw_hbm.at[ridx.at[slot4]], gbuf.at[slot2],
                              sem_gw.at[slot2]).wait()
        pltpu.make_async_copy(r1_hbm.at[pl.ds(0, _KB)], rbuf.at[slot2],
                              sem_r1.at[slot2]).wait()
        pltpu.make_async_copy(u4_hbm.at[0], ubuf.at[slot2],
                              sem_u.at[slot2]).wait()

    start_idx(0, 0)
    start_idx(1, 1)
    wait_idx(0)
    start_rows(0, 0, 0)

    def body(i, _):
        s2 = lax.rem(i, 2)
        s4 = lax.rem(i, 4)

        @pl.when(i + 1 < _NCHB)
        def _():
            wait_idx(lax.rem(i + 1, 4))
            start_rows(i + 1, lax.rem(i + 1, 2), lax.rem(i + 1, 4))

        @pl.when(i + 2 < _NCHB)
        def _():
            start_idx(i + 2, lax.rem(i + 2, 4))

        wait_rows(s2, s4)
        for g in range(_KB // 16):
            ev = lax.iota(jnp.int32, 16) + g * 16
            sl2 = jnp.full((16,), 0, jnp.int32) + s2
            acc = jnp.zeros((16,), jnp.float32)

            def ch(c, acc):
                cc = jnp.full((16,), 0, jnp.int32) + c
                hv = plsc.load_gather(hbuf, [sl2, ev, cc])
                rv = plsc.load_gather(rbuf, [sl2, ev, cc])
                gv = plsc.load_gather(gbuf, [sl2, ev, cc])
                return acc + (hv * rv) * gv

            acc = lax.fori_loop(0, _C, ch, acc, unroll=8)
            rv16 = ridx[s4, pl.ds(g * 16, 16)] * 4
            for c3 in range(3):
                uv = ubuf[s2, c3, pl.ds(g * 16, 16)]
                plsc.addupdate_scatter(facc, [rv16 + c3], acc * uv)
        return 0

    lax.fori_loop(0, _NCHB, body, 0, unroll=False)
    pltpu.sync_copy(facc, out_hbm.at[pl.ds(w * _NPAD * 4, _NPAD * 4)])


@functools.partial(
    pl.kernel,
    out_type=jax.ShapeDtypeStruct((_NW * _NPAD * 4,), jnp.float32),
    mesh=_sc_mesh,
    compiler_params=pltpu.CompilerParams(needs_layout_passes=False),
    scratch_types=[
        pltpu.VMEM((2, _KV), jnp.int32),
        pltpu.VMEM((2, _KV), jnp.int32),
        pltpu.VMEM((2, 4, _KV), jnp.float32),
        pltpu.VMEM((_NPAD * 4,), jnp.float32),
        pltpu.SemaphoreType.DMA((2,)),
        pltpu.SemaphoreType.DMA((2,)),
        pltpu.SemaphoreType.DMA((2,)),
        pltpu.SemaphoreType.DMA,
    ],
)
def _sc_forces_pass(gv_hbm, send_hbm, recv_hbm, z4_hbm, out_hbm,
                    sidx, ridx, gvbuf, facc, sem_s, sem_r, sem_g, sem0):
    cid = lax.axis_index("c")
    sid = lax.axis_index("s")
    w = cid * _NS + sid
    base = w * (_EP // _NW)

    pltpu.async_copy(z4_hbm, facc, sem0).wait()

    def start_chunk(c, slot):
        off = base + c * _KV
        pltpu.async_copy(send_hbm.at[pl.ds(off, _KV)], sidx.at[slot],
                         sem_s.at[slot])
        pltpu.async_copy(recv_hbm.at[pl.ds(off, _KV)], ridx.at[slot],
                         sem_r.at[slot])
        pltpu.async_copy(gv_hbm.at[base // _KV + c], gvbuf.at[slot],
                         sem_g.at[slot])

    def wait_chunk(slot):
        pltpu.make_async_copy(send_hbm.at[pl.ds(0, _KV)], sidx.at[slot],
                              sem_s.at[slot]).wait()
        pltpu.make_async_copy(recv_hbm.at[pl.ds(0, _KV)], ridx.at[slot],
                              sem_r.at[slot]).wait()
        pltpu.make_async_copy(gv_hbm.at[0], gvbuf.at[slot],
                              sem_g.at[slot]).wait()

    start_chunk(0, 0)

    def body(i, _):
        s2 = lax.rem(i, 2)

        @pl.when(i + 1 < _NCHV)
        def _():
            start_chunk(i + 1, lax.rem(i + 1, 2))

        wait_chunk(s2)
        for g in range(_KV // 16):
            ev = lax.iota(jnp.int32, 16) + g * 16
            sl2 = jnp.full((16,), 0, jnp.int32) + s2
            sv = sidx[s2, pl.ds(g * 16, 16)] * 4
            rv = ridx[s2, pl.ds(g * 16, 16)] * 4
            for c3 in range(3):
                gvv = gvbuf[s2, c3, pl.ds(g * 16, 16)]
                plsc.addupdate_scatter(facc, [sv + c3], gvv)
                plsc.addupdate_scatter(facc, [rv + c3], -gvv)
        return 0

    lax.fori_loop(0, _NCHV, body, 0, unroll=False)
    pltpu.sync_copy(facc, out_hbm.at[pl.ds(w * _NPAD * 4, _NPAD * 4)])


_BE = 1024


def _geom(v):
    ln = jnp.sqrt(jnp.sum(v * v, axis=1, keepdims=True) + 1e-12)
    inv = 1.0 / ln
    u = ln / _R_MAX
    Acf = 0.5 * (_P + 1) * (_P + 2)
    Bcf = _P * (_P + 2)
    Ccf = 0.5 * _P * (_P + 1)
    inside = u < 1.0
    fc = jnp.where(inside, 1.0 - Acf * u**_P + Bcf * u**(_P + 1)
                   - Ccf * u**(_P + 2), 0.0)
    dfc = jnp.where(inside, (-Acf * _P * u**(_P - 1) + Bcf * (_P + 1) * u**_P
                             - Ccf * (_P + 2) * u**(_P + 1)) / _R_MAX, 0.0)
    kk = (jax.lax.broadcasted_iota(jnp.int32, (1, _NB), 1) + 1
          ).astype(jnp.float32)
    arg = (kk * jnp.pi / _R_MAX) * ln
    sin_, cos_ = jnp.sin(arg), jnp.cos(arg)
    pref = jnp.float32((2.0 / _R_MAX) ** 0.5)
    bess = pref * sin_ * inv
    ef = bess * fc
    dbess = pref * ((kk * jnp.pi / _R_MAX) * cos_ * inv - sin_ * inv * inv)
    def_dl = dbess * fc + bess * dfc
    return ln, inv, ef, def_dl


def _dot(a, b):
    return jax.lax.dot_general(a, b, (((1,), (0,)), ((), ())),
                               preferred_element_type=jnp.float32,
                               precision=jax.lax.Precision.HIGHEST)


def _edge_fwd_kernel(vec_ref, w_ref_tree, u4_ref, r00_ref, r10_ref,
                     r01_ref, r11_ref):
    v = vec_ref[...]
    ln, inv, ef, _ = _geom(v)
    u4_ref[...] = v * inv
    outs = ((r00_ref, r10_ref), (r01_ref, r11_ref))
    for li in range(2):
        w1, w2, w3 = w_ref_tree[3 * li], w_ref_tree[3 * li + 1], w_ref_tree[3 * li + 2]
        r1 = _silu(_dot(ef, w1[...]))
        r2 = _silu(_dot(r1, w2[...]))
        r3 = _dot(r2, w3[...])
        outs[li][0][...] = r3[:, :_C]
        outs[li][1][...] = r3[:, _C:]


def _tc_edge_fwd(vec4, params):
    wl = []
    for lp in params["layers"]:
        wl += [lp["Wr1"], lp["Wr2"], lp["Wr3"]]
    nb = _EP // _BE
    full = lambda s: pl.BlockSpec(s, lambda i: tuple(0 for _ in s))
    outs = [jax.ShapeDtypeStruct((_EP, 4), jnp.float32)] + [
        jax.ShapeDtypeStruct((_EP, _C), jnp.float32)] * 4
    def kbody(vec_ref, *rest):
        wrefs = rest[:6]
        outr = rest[6:]
        _edge_fwd_kernel(vec_ref, wrefs, *outr)
    return pl.pallas_call(
        kbody,
        grid=(nb,),
        in_specs=[pl.BlockSpec((_BE, 4), lambda i: (i, 0))]
        + [full((_NB, 64)), full((64, 64)), full((64, 2 * _C))] * 2,
        out_specs=[pl.BlockSpec((_BE, 4), lambda i: (i, 0))]
        + [pl.BlockSpec((_BE, _C), lambda i: (i, 0))] * 4,
        out_shape=outs,
    )(vec4, *wl)


def _tc_edge_bwd(vec4, gr0, gr1, params):
    wl = []
    for lp in params["layers"]:
        wl += [lp["Wr1"], lp["Wr2"], lp["Wr3"]]
    nb = _EP // _BE
    full = lambda s: pl.BlockSpec(s, lambda i: tuple(0 for _ in s))

    def kbody(vec_ref, g0_ref, g1_ref, *rest, gv_ref):
        wrefs = rest
        v = vec_ref[...]
        ln, inv, ef, def_dl = _geom(v)
        gl = jnp.zeros((_BE, 1), jnp.float32)
        for li, gref in ((0, g0_ref), (1, g1_ref)):
            w1, w2, w3 = (wrefs[3 * li][...], wrefs[3 * li + 1][...],
                          wrefs[3 * li + 2][...])
            z1 = _dot(ef, w1)
            r1 = _silu(z1)
            z2 = _dot(r1, w2)
            r2 = _silu(z2)
            g_r2 = _dot(gref[...], w3[:, :_C].T)
            g_z2 = g_r2 * _dsilu(z2)
            g_r1 = _dot(g_z2, w2.T)
            g_z1 = g_r1 * _dsilu(z1)
            g_ef = _dot(g_z1, w1.T)
            gl = gl + jnp.sum(g_ef * def_dl, axis=1, keepdims=True)
        gv_ref[...] = gl * (v * inv)

    def kb(*refs):
        return kbody(*refs[:-1], gv_ref=refs[-1])

    return pl.pallas_call(
        kb,
        grid=(nb,),
        in_specs=[pl.BlockSpec((_BE, 4), lambda i: (i, 0)),
                  pl.BlockSpec((_BE, _C), lambda i: (i, 0)),
                  pl.BlockSpec((_BE, _C), lambda i: (i, 0))]
        + [full((_NB, 64)), full((64, 64)), full((64, 2 * _C))] * 2,
        out_specs=pl.BlockSpec((_BE, 4), lambda i: (i, 0)),
        out_shape=jax.ShapeDtypeStruct((_EP, 4), jnp.float32),
    )(vec4, gr0, gr1, *wl)


def _silu(x):
    return x * jax.nn.sigmoid(x)


def _dsilu(x):
    s = jax.nn.sigmoid(x)
    return s * (1 + x * (1 - s))


def _segsum_kernel(batch_ref, vals_ref, out_ref):
    i = pl.program_id(0)

    @pl.when(i == 0)
    def _init():
        out_ref[...] = jnp.zeros_like(out_ref)

    b = batch_ref[...]
    gids = jax.lax.broadcasted_iota(jnp.int32, (1, _GPAD), 1)
    onehot = (b == gids).astype(jnp.float32)
    out_ref[...] += jax.lax.dot_general(
        onehot, vals_ref[...], (((0,), (0,)), ((), ())),
        preferred_element_type=jnp.float32)


def _graph_segment_sums(batch, vals):
    K = vals.shape[1]
    BN = 2048
    nb = _NPAD // BN
    batch_p = jnp.full((_NPAD, 1), _GPAD - 1, jnp.int32).at[:_N, 0].set(batch.astype(jnp.int32))
    vals_p = jnp.zeros((_NPAD, K), jnp.float32).at[:_N].set(vals)
    out = pl.pallas_call(
        _segsum_kernel,
        grid=(nb,),
        in_specs=[
            pl.BlockSpec((BN, 1), lambda i: (i, 0)),
            pl.BlockSpec((BN, K), lambda i: (i, 0)),
        ],
        out_specs=pl.BlockSpec((_GPAD, K), lambda i: (0, 0)),
        out_shape=jax.ShapeDtypeStruct((_GPAD, K), jnp.float32),
    )(batch_p, vals_p)
    return out[:_G]


def kernel(positions, node_attrs, charges, params, edge_index, batch):
    send_p = jnp.zeros((_EP,), jnp.int32).at[:_E].set(
        edge_index[0].astype(jnp.int32))
    recv_p = jnp.zeros((_EP,), jnp.int32).at[:_E].set(
        edge_index[1].astype(jnp.int32))
    zeros4 = jnp.zeros((_NPAD * 4,), jnp.float32)
    vec = positions[recv_p] - positions[send_p]
    vec4 = jnp.zeros((_EP, 4), jnp.float32).at[:, :3].set(vec)
    vec4 = vec4.at[_E:, 0].set(3.0 * _R_MAX)

    u4, R00, R10, R01, R11 = _tc_edge_fwd(vec4, params)
    u4c = jnp.transpose(u4.reshape(_EP // 64, 64, 4), (0, 2, 1))

    node_e0 = node_attrs @ params["atomic_energies"]
    h0 = node_attrs @ params["W_embed"]

    h_in = h0
    saved = []
    he = []
    dparts = []
    for lp, R0, R1 in zip(params["layers"], (R00, R01), (R10, R11)):
        agg0 = _agg0_full(h_in, R0, send_p, recv_p) / _AVG_N
        h_out = h_in @ lp["Wsc"] + _silu(agg0)
        gate = _silu(agg0 @ lp["Wg"])
        gw = gate * lp["w_d"][None, :]
        dparts.append(_sc_dipole_pass(h_in, gw, R1, u4c, send_p, recv_p,
                                      zeros4))
        he.append(h_out @ lp["w_e"])
        saved.append(dict(R0=R0, h_in=h_in, agg0=agg0))
        h_in = h_out

    lp0, lp1 = params["layers"]
    sv0, sv1 = saved
    ga1 = lp1["w_e"][None, :] * _dsilu(sv1["agg0"]) / _AVG_N
    zc = jnp.zeros((_NROWS_SUB, _C), jnp.float32)
    g_R0_1, s1p = _bwd1_pass(sv1["h_in"], ga1, sv1["R0"], send_p, recv_p, zc)
    s1p = s1p.reshape(_NC, _NPAD, _C)
    g_hout0 = (lp0["w_e"][None, :] + (lp1["Wsc"] @ lp1["w_e"])[None, :]
               + s1p[0, :_N] + s1p[1, :_N])
    ga0 = g_hout0 * _dsilu(sv0["agg0"]) / _AVG_N
    g_R0_0 = _bwd0_pass(sv0["h_in"], ga0, send_p, recv_p)

    gv4 = _tc_edge_bwd(vec4, g_R0_0, g_R0_1, params)
    gv4c = jnp.transpose(gv4.reshape(_EP // 128, 128, 4), (0, 2, 1))
    fparts = _sc_forces_pass(gv4c, send_p, recv_p,
                             zeros4).reshape(_NW, _NPAD, 4)
    forces = jnp.sum(fparts, axis=0)[:_N, :3]

    dsum = (dparts[0] + dparts[1]).reshape(_NW, _NPAD, 4)
    atomic_dipoles = jnp.sum(dsum, axis=0)[:_N, :3] / _AVG_N

    vals = jnp.concatenate(
        [node_e0[:, None], he[0][:, None], he[1][:, None], atomic_dipoles,
         charges[:, None] * positions], axis=1)
    segs = _graph_segment_sums(batch, vals)
    e0, e1, e2 = segs[:, 0], segs[:, 1], segs[:, 2]
    total_dipole = segs[:, 3:6] + segs[:, 6:9]
    contributions = jnp.stack([e0, e1, e2], axis=-1)
    total_energy = e0 + e1 + e2
    return (total_energy, node_e0, contributions, forces, total_dipole, atomic_dipoles)


# speedup vs baseline: 1.0082x; 1.0082x over previous
import functools

import jax
import jax.numpy as jnp
from jax import lax
from jax.experimental import pallas as pl
from jax.experimental.pallas import tpu as pltpu
from jax.experimental.pallas import tpu_sc as plsc

_N = 10000
_E = 160000
_C = 128
_NB = 8
_G = 100
_R_MAX = 5.0
_P = 5
_AVG_N = 16.0

_NPAD = 10240
_GPAD = 128

_NC = 2
_NS = 16
_NW = _NC * _NS
_K = 128
_EP = 163840
_EPW = _EP // _NW
_NCHUNK = _EPW // _K
_NROWS_SUB = _NPAD // _NS

_sc_mesh = plsc.VectorSubcoreMesh(core_axis_name="c", subcore_axis_name="s")


def _edge_compute(k_edges, cw, fn):
    def rows(i, _):
        for rr in range(4):
            r = i * 4 + rr
            for j in range(cw // 16):
                fn(r, pl.ds(j * 16, 16))
        return 0

    lax.fori_loop(0, k_edges // 4, rows, 0, unroll=False)


def _make_edge_pass(has_ga, has_r0, scatter_sel, has_write, k_edges, cw, mslots=4):
    nchunk = _EP // _NW // k_edges
    outs = []
    if has_write:
        outs.append(jax.ShapeDtypeStruct((_EP, cw), jnp.float32))
    if scatter_sel is not None:
        outs.append(jax.ShapeDtypeStruct((_NC * _NPAD, cw), jnp.float32))
    scratch = []
    scratch.append(pltpu.VMEM((4, k_edges), jnp.int32))
    scratch.append(pltpu.SemaphoreType.DMA((4,)))
    if has_ga:
        scratch.append(pltpu.VMEM((4, k_edges), jnp.int32))
        scratch.append(pltpu.SemaphoreType.DMA((4,)))
    if scatter_sel is not None:
        scratch.append(pltpu.VMEM((mslots, k_edges), jnp.int32))
        scratch.append(pltpu.SemaphoreType.DMA((mslots,)))
    hslots = 4 if has_write else 2
    scratch.append(pltpu.VMEM((hslots, k_edges, cw), jnp.float32))
    scratch.append(pltpu.SemaphoreType.DMA((hslots,)))
    if has_ga:
        scratch.append(pltpu.VMEM((2, k_edges, cw), jnp.float32))
        scratch.append(pltpu.SemaphoreType.DMA((2,)))
    if has_r0:
        scratch.append(pltpu.VMEM((mslots, k_edges, cw), jnp.float32))
        scratch.append(pltpu.SemaphoreType.DMA((mslots,)))
    if has_write:
        scratch.append(pltpu.SemaphoreType.DMA((4,)))
    if scatter_sel is not None:
        scratch.append(pltpu.VMEM_SHARED((_NPAD, cw), jnp.float32))
        scratch.append(pltpu.SemaphoreType.DMA((mslots,)))

    @functools.partial(pl.kernel, out_type=tuple(outs) if len(outs) > 1
                       else outs[0], mesh=_sc_mesh, scratch_types=scratch,
                       compiler_params=pltpu.CompilerParams(
                           needs_layout_passes=False))
    def k(*refs):
        it = iter(refs)
        h_hbm = next(it)
        ga_hbm = next(it) if has_ga else None
        r0_hbm = next(it) if has_r0 else None
        send_hbm = next(it)
        recv_hbm = next(it)
        z_hbm = next(it) if scatter_sel is not None else None
        gr_hbm = next(it) if has_write else None
        out_hbm = next(it) if scatter_sel is not None else None
        hidx = next(it); sem_hidx = next(it)
        if has_ga:
            gaidx = next(it); sem_gaidx = next(it)
        if scatter_sel is not None:
            scidx = next(it); sem_scidx = next(it)
        hbuf = next(it); sem_h = next(it)
        if has_ga:
            gabuf = next(it); sem_ga = next(it)
        if has_r0:
            mbuf = next(it); sem_m = next(it)
        if has_write:
            sem_w = next(it)
        if scatter_sel is not None:
            acc = next(it); sem_sc = next(it)

        cid = lax.axis_index("c")
        sid = lax.axis_index("s")
        base = (cid * _NS + sid) * (_EP // _NW)
        sc_hbm = send_hbm if scatter_sel == "send" else recv_hbm

        if scatter_sel is not None:
            pltpu.sync_copy(z_hbm,
                            acc.at[pl.ds(sid * _NROWS_SUB, _NROWS_SUB)])
            plsc.subcore_barrier()

        def start_gidx(c, slot):
            off = base + c * k_edges
            pltpu.async_copy(send_hbm.at[pl.ds(off, k_edges)],
                             hidx.at[slot], sem_hidx.at[slot])
            if has_ga:
                pltpu.async_copy(recv_hbm.at[pl.ds(off, k_edges)],
                                 gaidx.at[slot], sem_gaidx.at[slot])

        def wait_gidx(slot):
            pltpu.make_async_copy(send_hbm.at[pl.ds(0, k_edges)],
                                  hidx.at[slot], sem_hidx.at[slot]).wait()
            if has_ga:
                pltpu.make_async_copy(recv_hbm.at[pl.ds(0, k_edges)],
                                      gaidx.at[slot], sem_gaidx.at[slot]).wait()

        def start_scidx(c, slot):
            off = base + c * k_edges
            pltpu.async_copy(sc_hbm.at[pl.ds(off, k_edges)],
                             scidx.at[slot], sem_scidx.at[slot])

        def start_rows(c, slot2, slot4, slotm):
            off = base + c * k_edges
            hs = slot4 if has_write else slot2
            pltpu.async_copy(h_hbm.at[hidx.at[slot4]], hbuf.at[hs],
                             sem_h.at[hs])
            if has_ga:
                pltpu.async_copy(ga_hbm.at[gaidx.at[slot4]], gabuf.at[slot2],
                                 sem_ga.at[slot2])
            if has_r0:
                pltpu.async_copy(r0_hbm.at[pl.ds(off, k_edges)],
                                 mbuf.at[slotm], sem_m.at[slotm])

        def wait_rows(slot2, slot4, slotm):
            hs = slot4 if has_write else slot2
            pltpu.make_async_copy(h_hbm.at[hidx.at[slot4]], hbuf.at[hs],
                                  sem_h.at[hs]).wait()
            if has_ga:
                pltpu.make_async_copy(ga_hbm.at[gaidx.at[slot4]],
                                      gabuf.at[slot2], sem_ga.at[slot2]).wait()
            if has_r0:
                pltpu.make_async_copy(r0_hbm.at[pl.ds(0, k_edges)],
                                      mbuf.at[slotm], sem_m.at[slotm]).wait()

        def wait_writes(slot4, slotm):
            if has_write:
                pltpu.make_async_copy(hbuf.at[slot4],
                                      gr_hbm.at[pl.ds(0, k_edges)],
                                      sem_w.at[slot4]).wait()
            if scatter_sel is not None:
                pltpu.make_async_copy(mbuf.at[slotm] if has_r0
                                      else hbuf.at[slot4],
                                      acc.at[scidx.at[slotm]],
                                      sem_sc.at[slotm]).wait()

        def compute(slot2, slot4, slotm):
            hs = slot4 if has_write else slot2
            if has_ga and has_r0:
                def fn(r, sl):
                    va = gabuf[slot2, r, sl]
                    hbuf[hs, r, sl] = va * hbuf[hs, r, sl]
                    mbuf[slotm, r, sl] = va * mbuf[slotm, r, sl]
            elif has_ga:
                def fn(r, sl):
                    hbuf[hs, r, sl] = gabuf[slot2, r, sl] * hbuf[hs, r, sl]
            else:
                def fn(r, sl):
                    mbuf[slotm, r, sl] = mbuf[slotm, r, sl] * hbuf[hs, r, sl]
            _edge_compute(k_edges, cw, fn)

        def start_writes(c, slot4, slotm):
            off = base + c * k_edges
            if has_write:
                pltpu.async_copy(hbuf.at[slot4], gr_hbm.at[pl.ds(off, k_edges)],
                                 sem_w.at[slot4])
            if scatter_sel is not None:
                pltpu.async_copy(mbuf.at[slotm] if has_r0 else hbuf.at[slot4],
                                 acc.at[scidx.at[slotm]], sem_sc.at[slotm],
                                 add=True)

        start_gidx(0, 0)
        start_gidx(1, 1)
        if scatter_sel is not None:
            start_scidx(0, 0)
        wait_gidx(0)
        start_rows(0, 0, 0, 0)

        def body(i, _):
            s2 = lax.rem(i, 2)
            s4 = lax.rem(i, 4)
            sm = lax.rem(i, mslots)
            n2 = lax.rem(i + 1, 2)
            n4 = lax.rem(i + 1, 4)
            nm = lax.rem(i + 1, mslots)

            @pl.when(jnp.logical_and(i + 1 < nchunk, i >= mslots - 1))
            def _():
                wait_writes(n4, nm)

            @pl.when(i + 1 < nchunk)
            def _():
                wait_gidx(n4)
                start_rows(i + 1, n2, n4, nm)

            @pl.when(i + 2 < nchunk)
            def _():
                start_gidx(i + 2, lax.rem(i + 2, 4))

            if scatter_sel is not None:
                @pl.when(i + 1 < nchunk)
                def _():
                    start_scidx(i + 1, nm)

            wait_rows(s2, s4, sm)
            compute(s2, s4, sm)
            if scatter_sel is not None:
                pltpu.make_async_copy(sc_hbm.at[pl.ds(0, k_edges)],
                                      scidx.at[sm], sem_scidx.at[sm]).wait()
            start_writes(i, s4, sm)
            return 0

        lax.fori_loop(0, nchunk, body, 0, unroll=False)
        for kk_ in range(max(0, nchunk - mslots), nchunk):
            wait_writes(kk_ % 4, kk_ % mslots)

        if scatter_sel is not None:
            plsc.subcore_barrier()
            pltpu.sync_copy(acc.at[pl.ds(sid * _NROWS_SUB, _NROWS_SUB)],
                            out_hbm.at[pl.ds(cid * _NPAD + sid * _NROWS_SUB,
                                             _NROWS_SUB)])

    return k


_agg0_pass = _make_edge_pass(False, True, "recv", False, 64, 128, mslots=3)
_bwd1_pass = _make_edge_pass(True, True, "send", True, 32, 128)
_bwd0_pass = _make_edge_pass(True, False, None, True, 128, 128)


def _agg0_full(h_tab, r0, send_p, recv_p):
    zc = jnp.zeros((_NROWS_SUB, _C), jnp.float32)
    p = _agg0_pass(h_tab, r0, send_p, recv_p, zc).reshape(_NC, _NPAD, _C)
    return p[0, :_N] + p[1, :_N]

_KV = 128
_NCHV = _EP // _NW // _KV


_KB = 64
_NCHB = _EP // _NW // _KB


@functools.partial(
    pl.kernel,
    out_type=jax.ShapeDtypeStruct((_NW * _NPAD * 4,), jnp.float32),
    mesh=_sc_mesh,
    compiler_params=pltpu.CompilerParams(needs_layout_passes=False),
    scratch_types=[
        pltpu.VMEM((4, _KB), jnp.int32),
        pltpu.VMEM((4, _KB), jnp.int32),
        pltpu.VMEM((2, _KB, _C), jnp.float32),
        pltpu.VMEM((2, _KB, _C), jnp.float32),
        pltpu.VMEM((2, _KB, _C), jnp.float32),
        pltpu.VMEM((2, 4, _KB), jnp.float32),
        pltpu.VMEM((_NPAD * 4,), jnp.float32),
        pltpu.SemaphoreType.DMA((4,)),
        pltpu.SemaphoreType.DMA((4,)),
        pltpu.SemaphoreType.DMA((2,)),
        pltpu.SemaphoreType.DMA((2,)),
        pltpu.SemaphoreType.DMA((2,)),
        pltpu.SemaphoreType.DMA((2,)),
        pltpu.SemaphoreType.DMA,
    ],
)
def _sc_dipole_pass(h_hbm, gw_hbm, r1_hbm, u4_hbm, send_hbm, recv_hbm,
                    z4_hbm, out_hbm, sidx, ridx, hbuf, rbuf, gbuf, ubuf,
                    facc, sem_si, sem_ri, sem_h, sem_r1, sem_gw, sem_u, sem0):
    cid = lax.axis_index("c")
    sid = lax.axis_index("s")
    w = cid * _NS + sid
    base = w * (_EP // _NW)

    pltpu.async_copy(z4_hbm, facc, sem0).wait()

    def start_idx(c, slot):
        off = base + c * _KB
        pltpu.async_copy(send_hbm.at[pl.ds(off, _KB)], sidx.at[slot],
                         sem_si.at[slot])
        pltpu.async_copy(recv_hbm.at[pl.ds(off, _KB)], ridx.at[slot],
                         sem_ri.at[slot])

    def wait_idx(slot):
        pltpu.make_async_copy(send_hbm.at[pl.ds(0, _KB)], sidx.at[slot],
                              sem_si.at[slot]).wait()
        pltpu.make_async_copy(recv_hbm.at[pl.ds(0, _KB)], ridx.at[slot],
                              sem_ri.at[slot]).wait()

    def start_rows(c, slot2, slot4):
        off = base + c * _KB
        pltpu.async_copy(h_hbm.at[sidx.at[slot4]], hbuf.at[slot2],
                         sem_h.at[slot2])
        pltpu.async_copy(gw_hbm.at[ridx.at[slot4]], gbuf.at[slot2],
                         sem_gw.at[slot2])
        pltpu.async_copy(r1_hbm.at[pl.ds(off, _KB)], rbuf.at[slot2],
                         sem_r1.at[slot2])
        pltpu.async_copy(u4_hbm.at[base // _KB + c], ubuf.at[slot2],
                         sem_u.at[slot2])

    def wait_rows(slot2, slot4):
        pltpu.make_async_copy(h_hbm.at[sidx.at[slot4]], hbuf.at[slot2],
                              sem_h.at[slot2]).wait()
        pltpu.make_async_copy(gw_hbm.at[ridx.at[slot4]], gbuf.at[slot2],
                              sem_gw.at[slot2]).wait()
        pltpu.make_async_copy(r1_hbm.at[pl.ds(0, _KB)], rbuf.at[slot2],
                              sem_r1.at[slot2]).wait()
        pltpu.make_async_copy(u4_hbm.at[0], ubuf.at[slot2],
                              sem_u.at[slot2]).wait()

    start_idx(0, 0)
    start_idx(1, 1)
    wait_idx(0)
    start_rows(0, 0, 0)

    def body(i, _):
        s2 = lax.rem(i, 2)
        s4 = lax.rem(i, 4)

        @pl.when(i + 1 < _NCHB)
        def _():
            wait_idx(lax.rem(i + 1, 4))
            start_rows(i + 1, lax.rem(i + 1, 2), lax.rem(i + 1, 4))

        @pl.when(i + 2 < _NCHB)
        def _():
            start_idx(i + 2, lax.rem(i + 2, 4))

        wait_rows(s2, s4)
        for g in range(_KB // 16):
            ev = lax.iota(jnp.int32, 16) + g * 16
            sl2 = jnp.full((16,), 0, jnp.int32) + s2
            acc = jnp.zeros((16,), jnp.float32)

            def ch(c, acc):
                cc = jnp.full((16,), 0, jnp.int32) + c
                hv = plsc.load_gather(hbuf, [sl2, ev, cc])
                rv = plsc.load_gather(rbuf, [sl2, ev, cc])
                gv = plsc.load_gather(gbuf, [sl2, ev, cc])
                return acc + (hv * rv) * gv

            acc = lax.fori_loop(0, _C, ch, acc, unroll=8)
            rv16 = ridx[s4, pl.ds(g * 16, 16)] * 4
            for c3 in range(3):
                uv = ubuf[s2, c3, pl.ds(g * 16, 16)]
                plsc.addupdate_scatter(facc, [rv16 + c3], acc * uv)
        return 0

    lax.fori_loop(0, _NCHB, body, 0, unroll=False)
    pltpu.sync_copy(facc, out_hbm.at[pl.ds(w * _NPAD * 4, _NPAD * 4)])


@functools.partial(
    pl.kernel,
    out_type=jax.ShapeDtypeStruct((_NW * _NPAD * 4,), jnp.float32),
    mesh=_sc_mesh,
    compiler_params=pltpu.CompilerParams(needs_layout_passes=False),
    scratch_types=[
        pltpu.VMEM((2, _KV), jnp.int32),
        pltpu.VMEM((2, _KV), jnp.int32),
        pltpu.VMEM((2, 4, _KV), jnp.float32),
        pltpu.VMEM((_NPAD * 4,), jnp.float32),
        pltpu.SemaphoreType.DMA((2,)),
        pltpu.SemaphoreType.DMA((2,)),
        pltpu.SemaphoreType.DMA((2,)),
        pltpu.SemaphoreType.DMA,
    ],
)
def _sc_forces_pass(gv_hbm, send_hbm, recv_hbm, z4_hbm, out_hbm,
                    sidx, ridx, gvbuf, facc, sem_s, sem_r, sem_g, sem0):
    cid = lax.axis_index("c")
    sid = lax.axis_index("s")
    w = cid * _NS + sid
    base = w * (_EP // _NW)

    pltpu.async_copy(z4_hbm, facc, sem0).wait()

    def start_chunk(c, slot):
        off = base + c * _KV
        pltpu.async_copy(send_hbm.at[pl.ds(off, _KV)], sidx.at[slot],
                         sem_s.at[slot])
        pltpu.async_copy(recv_hbm.at[pl.ds(off, _KV)], ridx.at[slot],
                         sem_r.at[slot])
        pltpu.async_copy(gv_hbm.at[base // _KV + c], gvbuf.at[slot],
                         sem_g.at[slot])

    def wait_chunk(slot):
        pltpu.make_async_copy(send_hbm.at[pl.ds(0, _KV)], sidx.at[slot],
                              sem_s.at[slot]).wait()
        pltpu.make_async_copy(recv_hbm.at[pl.ds(0, _KV)], ridx.at[slot],
                              sem_r.at[slot]).wait()
        pltpu.make_async_copy(gv_hbm.at[0], gvbuf.at[slot],
                              sem_g.at[slot]).wait()

    start_chunk(0, 0)

    def body(i, _):
        s2 = lax.rem(i, 2)

        @pl.when(i + 1 < _NCHV)
        def _():
            start_chunk(i + 1, lax.rem(i + 1, 2))

        wait_chunk(s2)
        for g in range(_KV // 16):
            ev = lax.iota(jnp.int32, 16) + g * 16
            sl2 = jnp.full((16,), 0, jnp.int32) + s2
            sv = sidx[s2, pl.ds(g * 16, 16)] * 4
            rv = ridx[s2, pl.ds(g * 16, 16)] * 4
            for c3 in range(3):
                gvv = gvbuf[s2, c3, pl.ds(g * 16, 16)]
                plsc.addupdate_scatter(facc, [sv + c3], gvv)
                plsc.addupdate_scatter(facc, [rv + c3], -gvv)
        return 0

    lax.fori_loop(0, _NCHV, body, 0, unroll=False)
    pltpu.sync_copy(facc, out_hbm.at[pl.ds(w * _NPAD * 4, _NPAD * 4)])


_BE = 2048


def _geom(v):
    ln = jnp.sqrt(jnp.sum(v * v, axis=1, keepdims=True) + 1e-12)
    inv = 1.0 / ln
    u = ln / _R_MAX
    Acf = 0.5 * (_P + 1) * (_P + 2)
    Bcf = _P * (_P + 2)
    Ccf = 0.5 * _P * (_P + 1)
    inside = u < 1.0
    fc = jnp.where(inside, 1.0 - Acf * u**_P + Bcf * u**(_P + 1)
                   - Ccf * u**(_P + 2), 0.0)
    dfc = jnp.where(inside, (-Acf * _P * u**(_P - 1) + Bcf * (_P + 1) * u**_P
                             - Ccf * (_P + 2) * u**(_P + 1)) / _R_MAX, 0.0)
    kk = (jax.lax.broadcasted_iota(jnp.int32, (1, _NB), 1) + 1
          ).astype(jnp.float32)
    arg = (kk * jnp.pi / _R_MAX) * ln
    sin_, cos_ = jnp.sin(arg), jnp.cos(arg)
    pref = jnp.float32((2.0 / _R_MAX) ** 0.5)
    bess = pref * sin_ * inv
    ef = bess * fc
    dbess = pref * ((kk * jnp.pi / _R_MAX) * cos_ * inv - sin_ * inv * inv)
    def_dl = dbess * fc + bess * dfc
    return ln, inv, ef, def_dl


def _dot(a, b):
    return jax.lax.dot_general(a, b, (((1,), (0,)), ((), ())),
                               preferred_element_type=jnp.float32,
                               precision=jax.lax.Precision.HIGHEST)


def _edge_fwd_kernel(vec_ref, w_ref_tree, u4_ref, r00_ref, r10_ref,
                     r01_ref, r11_ref):
    v = vec_ref[...]
    ln, inv, ef, _ = _geom(v)
    u4_ref[...] = v * inv
    outs = ((r00_ref, r10_ref), (r01_ref, r11_ref))
    for li in range(2):
        w1, w2, w3 = w_ref_tree[3 * li], w_ref_tree[3 * li + 1], w_ref_tree[3 * li + 2]
        r1 = _silu(_dot(ef, w1[...]))
        r2 = _silu(_dot(r1, w2[...]))
        r3 = _dot(r2, w3[...])
        outs[li][0][...] = r3[:, :_C]
        outs[li][1][...] = r3[:, _C:]


def _tc_edge_fwd(vec4, params):
    wl = []
    for lp in params["layers"]:
        wl += [lp["Wr1"], lp["Wr2"], lp["Wr3"]]
    nb = _EP // _BE
    full = lambda s: pl.BlockSpec(s, lambda i: tuple(0 for _ in s))
    outs = [jax.ShapeDtypeStruct((_EP, 4), jnp.float32)] + [
        jax.ShapeDtypeStruct((_EP, _C), jnp.float32)] * 4
    def kbody(vec_ref, *rest):
        wrefs = rest[:6]
        outr = rest[6:]
        _edge_fwd_kernel(vec_ref, wrefs, *outr)
    return pl.pallas_call(
        kbody,
        grid=(nb,),
        in_specs=[pl.BlockSpec((_BE, 4), lambda i: (i, 0))]
        + [full((_NB, 64)), full((64, 64)), full((64, 2 * _C))] * 2,
        out_specs=[pl.BlockSpec((_BE, 4), lambda i: (i, 0))]
        + [pl.BlockSpec((_BE, _C), lambda i: (i, 0))] * 4,
        out_shape=outs,
    )(vec4, *wl)


def _tc_edge_bwd(vec4, gr0, gr1, params):
    wl = []
    for lp in params["layers"]:
        wl += [lp["Wr1"], lp["Wr2"], lp["Wr3"]]
    nb = _EP // _BE
    full = lambda s: pl.BlockSpec(s, lambda i: tuple(0 for _ in s))

    def kbody(vec_ref, g0_ref, g1_ref, *rest, gv_ref):
        wrefs = rest
        v = vec_ref[...]
        ln, inv, ef, def_dl = _geom(v)
        gl = jnp.zeros((_BE, 1), jnp.float32)
        for li, gref in ((0, g0_ref), (1, g1_ref)):
            w1, w2, w3 = (wrefs[3 * li][...], wrefs[3 * li + 1][...],
                          wrefs[3 * li + 2][...])
            z1 = _dot(ef, w1)
            r1 = _silu(z1)
            z2 = _dot(r1, w2)
            r2 = _silu(z2)
            g_r2 = _dot(gref[...], w3[:, :_C].T)
            g_z2 = g_r2 * _dsilu(z2)
            g_r1 = _dot(g_z2, w2.T)
            g_z1 = g_r1 * _dsilu(z1)
            g_ef = _dot(g_z1, w1.T)
            gl = gl + jnp.sum(g_ef * def_dl, axis=1, keepdims=True)
        gv_ref[...] = gl * (v * inv)

    def kb(*refs):
        return kbody(*refs[:-1], gv_ref=refs[-1])

    return pl.pallas_call(
        kb,
        grid=(nb,),
        in_specs=[pl.BlockSpec((_BE, 4), lambda i: (i, 0)),
                  pl.BlockSpec((_BE, _C), lambda i: (i, 0)),
                  pl.BlockSpec((_BE, _C), lambda i: (i, 0))]
        + [full((_NB, 64)), full((64, 64)), full((64, 2 * _C))] * 2,
        out_specs=pl.BlockSpec((_BE, 4), lambda i: (i, 0)),
        out_shape=jax.ShapeDtypeStruct((_EP, 4), jnp.float32),
    )(vec4, gr0, gr1, *wl)


def _silu(x):
    return x * jax.nn.sigmoid(x)


def _dsilu(x):
    s = jax.nn.sigmoid(x)
    return s * (1 + x * (1 - s))


def _segsum_kernel(batch_ref, vals_ref, out_ref):
    i = pl.program_id(0)

    @pl.when(i == 0)
    def _init():
        out_ref[...] = jnp.zeros_like(out_ref)

    b = batch_ref[...]
    gids = jax.lax.broadcasted_iota(jnp.int32, (1, _GPAD), 1)
    onehot = (b == gids).astype(jnp.float32)
    out_ref[...] += jax.lax.dot_general(
        onehot, vals_ref[...], (((0,), (0,)), ((), ())),
        preferred_element_type=jnp.float32)


def _graph_segment_sums(batch, vals):
    K = vals.shape[1]
    BN = 2048
    nb = _NPAD // BN
    batch_p = jnp.full((_NPAD, 1), _GPAD - 1, jnp.int32).at[:_N, 0].set(batch.astype(jnp.int32))
    vals_p = jnp.zeros((_NPAD, K), jnp.float32).at[:_N].set(vals)
    out = pl.pallas_call(
        _segsum_kernel,
        grid=(nb,),
        in_specs=[
            pl.BlockSpec((BN, 1), lambda i: (i, 0)),
            pl.BlockSpec((BN, K), lambda i: (i, 0)),
        ],
        out_specs=pl.BlockSpec((_GPAD, K), lambda i: (0, 0)),
        out_shape=jax.ShapeDtypeStruct((_GPAD, K), jnp.float32),
    )(batch_p, vals_p)
    return out[:_G]


def kernel(positions, node_attrs, charges, params, edge_index, batch):
    send_p = jnp.zeros((_EP,), jnp.int32).at[:_E].set(
        edge_index[0].astype(jnp.int32))
    recv_p = jnp.zeros((_EP,), jnp.int32).at[:_E].set(
        edge_index[1].astype(jnp.int32))
    zeros4 = jnp.zeros((_NPAD * 4,), jnp.float32)
    vec = positions[recv_p] - positions[send_p]
    vec4 = jnp.zeros((_EP, 4), jnp.float32).at[:, :3].set(vec)
    vec4 = vec4.at[_E:, 0].set(3.0 * _R_MAX)

    u4, R00, R10, R01, R11 = _tc_edge_fwd(vec4, params)
    u4c = jnp.transpose(u4.reshape(_EP // 64, 64, 4), (0, 2, 1))

    node_e0 = node_attrs @ params["atomic_energies"]
    h0 = node_attrs @ params["W_embed"]

    h_in = h0
    saved = []
    he = []
    dparts = []
    for lp, R0, R1 in zip(params["layers"], (R00, R01), (R10, R11)):
        agg0 = _agg0_full(h_in, R0, send_p, recv_p) / _AVG_N
        h_out = h_in @ lp["Wsc"] + _silu(agg0)
        gate = _silu(agg0 @ lp["Wg"])
        gw = gate * lp["w_d"][None, :]
        dparts.append(_sc_dipole_pass(h_in, gw, R1, u4c, send_p, recv_p,
                                      zeros4))
        he.append(h_out @ lp["w_e"])
        saved.append(dict(R0=R0, h_in=h_in, agg0=agg0))
        h_in = h_out

    lp0, lp1 = params["layers"]
    sv0, sv1 = saved
    ga1 = lp1["w_e"][None, :] * _dsilu(sv1["agg0"]) / _AVG_N
    zc = jnp.zeros((_NROWS_SUB, _C), jnp.float32)
    g_R0_1, s1p = _bwd1_pass(sv1["h_in"], ga1, sv1["R0"], send_p, recv_p, zc)
    s1p = s1p.reshape(_NC, _NPAD, _C)
    g_hout0 = (lp0["w_e"][None, :] + (lp1["Wsc"] @ lp1["w_e"])[None, :]
               + s1p[0, :_N] + s1p[1, :_N])
    ga0 = g_hout0 * _dsilu(sv0["agg0"]) / _AVG_N
    g_R0_0 = _bwd0_pass(sv0["h_in"], ga0, send_p, recv_p)

    gv4 = _tc_edge_bwd(vec4, g_R0_0, g_R0_1, params)
    gv4c = jnp.transpose(gv4.reshape(_EP // 128, 128, 4), (0, 2, 1))
    fparts = _sc_forces_pass(gv4c, send_p, recv_p,
                             zeros4).reshape(_NW, _NPAD, 4)
    forces = jnp.sum(fparts, axis=0)[:_N, :3]

    dsum = (dparts[0] + dparts[1]).reshape(_NW, _NPAD, 4)
    atomic_dipoles = jnp.sum(dsum, axis=0)[:_N, :3] / _AVG_N

    vals = jnp.concatenate(
        [node_e0[:, None], he[0][:, None], he[1][:, None], atomic_dipoles,
         charges[:, None] * positions], axis=1)
    segs = _graph_segment_sums(batch, vals)
    e0, e1, e2 = segs[:, 0], segs[:, 1], segs[:, 2]
    total_dipole = segs[:, 3:6] + segs[:, 6:9]
    contributions = jnp.stack([e0, e1, e2], axis=-1)
    total_energy = e0 + e1 + e2
    return (total_energy, node_e0, contributions, forces, total_dipole, atomic_dipoles)


# speedup vs baseline: 1.1338x; 1.1246x over previous
import functools

import jax
import jax.numpy as jnp
from jax import lax
from jax.experimental import pallas as pl
from jax.experimental.pallas import tpu as pltpu
from jax.experimental.pallas import tpu_sc as plsc

_N = 10000
_E = 160000
_C = 128
_NB = 8
_G = 100
_R_MAX = 5.0
_P = 5
_AVG_N = 16.0

_NPAD = 10240
_GPAD = 128

_NC = 2
_NS = 16
_NW = _NC * _NS
_K = 128
_EP = 163840
_EPW = _EP // _NW
_NCHUNK = _EPW // _K
_NROWS_SUB = _NPAD // _NS

_sc_mesh = plsc.VectorSubcoreMesh(core_axis_name="c", subcore_axis_name="s")


def _edge_compute(k_edges, cw, fn):
    def rows(i, _):
        for rr in range(4):
            r = i * 4 + rr
            for j in range(cw // 16):
                fn(r, pl.ds(j * 16, 16))
        return 0

    lax.fori_loop(0, k_edges // 4, rows, 0, unroll=False)


def _make_edge_pass(has_ga, has_r0, scatter_sel, has_write, k_edges, cw, mslots=4):
    nchunk = _EP // _NW // k_edges
    outs = []
    if has_write:
        outs.append(jax.ShapeDtypeStruct((_EP, cw), jnp.float32))
    if scatter_sel is not None:
        outs.append(jax.ShapeDtypeStruct((_NC * _NPAD, cw), jnp.float32))
    scratch = []
    scratch.append(pltpu.VMEM((4, k_edges), jnp.int32))
    scratch.append(pltpu.SemaphoreType.DMA((4,)))
    if has_ga:
        scratch.append(pltpu.VMEM((4, k_edges), jnp.int32))
        scratch.append(pltpu.SemaphoreType.DMA((4,)))
    if scatter_sel is not None:
        scratch.append(pltpu.VMEM((mslots, k_edges), jnp.int32))
        scratch.append(pltpu.SemaphoreType.DMA((mslots,)))
    hslots = 4 if has_write else 2
    scratch.append(pltpu.VMEM((hslots, k_edges, cw), jnp.float32))
    scratch.append(pltpu.SemaphoreType.DMA((hslots,)))
    if has_ga:
        scratch.append(pltpu.VMEM((2, k_edges, cw), jnp.float32))
        scratch.append(pltpu.SemaphoreType.DMA((2,)))
    if has_r0:
        scratch.append(pltpu.VMEM((mslots, k_edges, cw), jnp.float32))
        scratch.append(pltpu.SemaphoreType.DMA((mslots,)))
    if has_write:
        scratch.append(pltpu.SemaphoreType.DMA((4,)))
    if scatter_sel is not None:
        scratch.append(pltpu.VMEM_SHARED((_NPAD, cw), jnp.float32))
        scratch.append(pltpu.SemaphoreType.DMA((mslots,)))

    @functools.partial(pl.kernel, out_type=tuple(outs) if len(outs) > 1
                       else outs[0], mesh=_sc_mesh, scratch_types=scratch,
                       compiler_params=pltpu.CompilerParams(
                           needs_layout_passes=False))
    def k(*refs):
        it = iter(refs)
        h_hbm = next(it)
        ga_hbm = next(it) if has_ga else None
        r0_hbm = next(it) if has_r0 else None
        send_hbm = next(it)
        recv_hbm = next(it)
        z_hbm = next(it) if scatter_sel is not None else None
        gr_hbm = next(it) if has_write else None
        out_hbm = next(it) if scatter_sel is not None else None
        hidx = next(it); sem_hidx = next(it)
        if has_ga:
            gaidx = next(it); sem_gaidx = next(it)
        if scatter_sel is not None:
            scidx = next(it); sem_scidx = next(it)
        hbuf = next(it); sem_h = next(it)
        if has_ga:
            gabuf = next(it); sem_ga = next(it)
        if has_r0:
            mbuf = next(it); sem_m = next(it)
        if has_write:
            sem_w = next(it)
        if scatter_sel is not None:
            acc = next(it); sem_sc = next(it)

        cid = lax.axis_index("c")
        sid = lax.axis_index("s")
        base = (cid * _NS + sid) * (_EP // _NW)
        sc_hbm = send_hbm if scatter_sel == "send" else recv_hbm

        if scatter_sel is not None:
            pltpu.sync_copy(z_hbm,
                            acc.at[pl.ds(sid * _NROWS_SUB, _NROWS_SUB)])
            plsc.subcore_barrier()

        def start_gidx(c, slot):
            off = base + c * k_edges
            pltpu.async_copy(send_hbm.at[pl.ds(off, k_edges)],
                             hidx.at[slot], sem_hidx.at[slot])
            if has_ga:
                pltpu.async_copy(recv_hbm.at[pl.ds(off, k_edges)],
                                 gaidx.at[slot], sem_gaidx.at[slot])

        def wait_gidx(slot):
            pltpu.make_async_copy(send_hbm.at[pl.ds(0, k_edges)],
                                  hidx.at[slot], sem_hidx.at[slot]).wait()
            if has_ga:
                pltpu.make_async_copy(recv_hbm.at[pl.ds(0, k_edges)],
                                      gaidx.at[slot], sem_gaidx.at[slot]).wait()

        def start_scidx(c, slot):
            off = base + c * k_edges
            pltpu.async_copy(sc_hbm.at[pl.ds(off, k_edges)],
                             scidx.at[slot], sem_scidx.at[slot])

        def start_rows(c, slot2, slot4, slotm):
            off = base + c * k_edges
            hs = slot4 if has_write else slot2
            pltpu.async_copy(h_hbm.at[hidx.at[slot4]], hbuf.at[hs],
                             sem_h.at[hs])
            if has_ga:
                pltpu.async_copy(ga_hbm.at[gaidx.at[slot4]], gabuf.at[slot2],
                                 sem_ga.at[slot2])
            if has_r0:
                pltpu.async_copy(r0_hbm.at[pl.ds(off, k_edges)],
                                 mbuf.at[slotm], sem_m.at[slotm])

        def wait_rows(slot2, slot4, slotm):
            hs = slot4 if has_write else slot2
            pltpu.make_async_copy(h_hbm.at[hidx.at[slot4]], hbuf.at[hs],
                                  sem_h.at[hs]).wait()
            if has_ga:
                pltpu.make_async_copy(ga_hbm.at[gaidx.at[slot4]],
                                      gabuf.at[slot2], sem_ga.at[slot2]).wait()
            if has_r0:
                pltpu.make_async_copy(r0_hbm.at[pl.ds(0, k_edges)],
                                      mbuf.at[slotm], sem_m.at[slotm]).wait()

        def wait_writes(slot4, slotm):
            if has_write:
                pltpu.make_async_copy(hbuf.at[slot4],
                                      gr_hbm.at[pl.ds(0, k_edges)],
                                      sem_w.at[slot4]).wait()
            if scatter_sel is not None:
                pltpu.make_async_copy(mbuf.at[slotm] if has_r0
                                      else hbuf.at[slot4],
                                      acc.at[scidx.at[slotm]],
                                      sem_sc.at[slotm]).wait()

        def compute(slot2, slot4, slotm):
            hs = slot4 if has_write else slot2
            if has_ga and has_r0:
                def fn(r, sl):
                    va = gabuf[slot2, r, sl]
                    hbuf[hs, r, sl] = va * hbuf[hs, r, sl]
                    mbuf[slotm, r, sl] = va * mbuf[slotm, r, sl]
            elif has_ga:
                def fn(r, sl):
                    hbuf[hs, r, sl] = gabuf[slot2, r, sl] * hbuf[hs, r, sl]
            else:
                def fn(r, sl):
                    mbuf[slotm, r, sl] = mbuf[slotm, r, sl] * hbuf[hs, r, sl]
            _edge_compute(k_edges, cw, fn)

        def start_writes(c, slot4, slotm):
            off = base + c * k_edges
            if has_write:
                pltpu.async_copy(hbuf.at[slot4], gr_hbm.at[pl.ds(off, k_edges)],
                                 sem_w.at[slot4])
            if scatter_sel is not None:
                pltpu.async_copy(mbuf.at[slotm] if has_r0 else hbuf.at[slot4],
                                 acc.at[scidx.at[slotm]], sem_sc.at[slotm],
                                 add=True)

        start_gidx(0, 0)
        start_gidx(1, 1)
        if scatter_sel is not None:
            start_scidx(0, 0)
        wait_gidx(0)
        start_rows(0, 0, 0, 0)

        def body(i, _):
            s2 = lax.rem(i, 2)
            s4 = lax.rem(i, 4)
            sm = lax.rem(i, mslots)
            n2 = lax.rem(i + 1, 2)
            n4 = lax.rem(i + 1, 4)
            nm = lax.rem(i + 1, mslots)

            @pl.when(jnp.logical_and(i + 1 < nchunk, i >= mslots - 1))
            def _():
                wait_writes(n4, nm)

            @pl.when(i + 1 < nchunk)
            def _():
                wait_gidx(n4)
                start_rows(i + 1, n2, n4, nm)

            @pl.when(i + 2 < nchunk)
            def _():
                start_gidx(i + 2, lax.rem(i + 2, 4))

            if scatter_sel is not None:
                @pl.when(i + 1 < nchunk)
                def _():
                    start_scidx(i + 1, nm)

            wait_rows(s2, s4, sm)
            compute(s2, s4, sm)
            if scatter_sel is not None:
                pltpu.make_async_copy(sc_hbm.at[pl.ds(0, k_edges)],
                                      scidx.at[sm], sem_scidx.at[sm]).wait()
            start_writes(i, s4, sm)
            return 0

        lax.fori_loop(0, nchunk, body, 0, unroll=False)
        for kk_ in range(max(0, nchunk - mslots), nchunk):
            wait_writes(kk_ % 4, kk_ % mslots)

        if scatter_sel is not None:
            plsc.subcore_barrier()
            pltpu.sync_copy(acc.at[pl.ds(sid * _NROWS_SUB, _NROWS_SUB)],
                            out_hbm.at[pl.ds(cid * _NPAD + sid * _NROWS_SUB,
                                             _NROWS_SUB)])

    return k


_agg0_pass = _make_edge_pass(False, True, "recv", False, 64, 128, mslots=3)
_bwd1_pass = _make_edge_pass(True, True, "send", True, 32, 128)
_bwd0_pass = _make_edge_pass(True, False, None, True, 128, 128)


def _agg0_full(h_tab, r0, send_p, recv_p):
    zc = jnp.zeros((_NROWS_SUB, _C), jnp.float32)
    p = _agg0_pass(h_tab, r0, send_p, recv_p, zc).reshape(_NC, _NPAD, _C)
    return p[0, :_N] + p[1, :_N]

_KV = 128
_NCHV = _EP // _NW // _KV


_KB = 64
_NCHB = _EP // _NW // _KB


@functools.partial(
    pl.kernel,
    out_type=jax.ShapeDtypeStruct((_NW * _NPAD * 4,), jnp.float32),
    mesh=_sc_mesh,
    compiler_params=pltpu.CompilerParams(needs_layout_passes=False),
    scratch_types=[
        pltpu.VMEM((4, _KB), jnp.int32),
        pltpu.VMEM((4, _KB), jnp.int32),
        pltpu.VMEM((2, _KB, _C), jnp.float32),
        pltpu.VMEM((2, _KB, _C), jnp.float32),
        pltpu.VMEM((2, _KB, _C), jnp.float32),
        pltpu.VMEM((2, 4, _KB), jnp.float32),
        pltpu.VMEM((_NPAD * 4,), jnp.float32),
        pltpu.SemaphoreType.DMA((4,)),
        pltpu.SemaphoreType.DMA((4,)),
        pltpu.SemaphoreType.DMA((2,)),
        pltpu.SemaphoreType.DMA((2,)),
        pltpu.SemaphoreType.DMA((2,)),
        pltpu.SemaphoreType.DMA((2,)),
        pltpu.SemaphoreType.DMA,
    ],
)
def _sc_dipole_pass(h_hbm, gw_hbm, r1_hbm, u4_hbm, send_hbm, recv_hbm,
                    z4_hbm, out_hbm, sidx, ridx, hbuf, rbuf, gbuf, ubuf,
                    facc, sem_si, sem_ri, sem_h, sem_r1, sem_gw, sem_u, sem0):
    cid = lax.axis_index("c")
    sid = lax.axis_index("s")
    w = cid * _NS + sid
    base = w * (_EP // _NW)

    pltpu.async_copy(z4_hbm, facc, sem0).wait()

    def start_idx(c, slot):
        off = base + c * _KB
        pltpu.async_copy(send_hbm.at[pl.ds(off, _KB)], sidx.at[slot],
                         sem_si.at[slot])
        pltpu.async_copy(recv_hbm.at[pl.ds(off, _KB)], ridx.at[slot],
                         sem_ri.at[slot])

    def wait_idx(slot):
        pltpu.make_async_copy(send_hbm.at[pl.ds(0, _KB)], sidx.at[slot],
                              sem_si.at[slot]).wait()
        pltpu.make_async_copy(recv_hbm.at[pl.ds(0, _KB)], ridx.at[slot],
                              sem_ri.at[slot]).wait()

    def start_rows(c, slot2, slot4):
        off = base + c * _KB
        pltpu.async_copy(h_hbm.at[sidx.at[slot4]], hbuf.at[slot2],
                         sem_h.at[slot2])
        pltpu.async_copy(gw_hbm.at[ridx.at[slot4]], gbuf.at[slot2],
                         sem_gw.at[slot2])
        pltpu.async_copy(r1_hbm.at[pl.ds(off, _KB)], rbuf.at[slot2],
                         sem_r1.at[slot2])
        pltpu.async_copy(u4_hbm.at[base // _KB + c], ubuf.at[slot2],
                         sem_u.at[slot2])

    def wait_rows(slot2, slot4):
        pltpu.make_async_copy(h_hbm.at[sidx.at[slot4]], hbuf.at[slot2],
                              sem_h.at[slot2]).wait()
        pltpu.make_async_copy(gw_hbm.at[ridx.at[slot4]], gbuf.at[slot2],
                              sem_gw.at[slot2]).wait()
        pltpu.make_async_copy(r1_hbm.at[pl.ds(0, _KB)], rbuf.at[slot2],
                              sem_r1.at[slot2]).wait()
        pltpu.make_async_copy(u4_hbm.at[0], ubuf.at[slot2],
                              sem_u.at[slot2]).wait()

    start_idx(0, 0)
    start_idx(1, 1)
    wait_idx(0)
    start_rows(0, 0, 0)

    def body(i, _):
        s2 = lax.rem(i, 2)
        s4 = lax.rem(i, 4)

        @pl.when(i + 1 < _NCHB)
        def _():
            wait_idx(lax.rem(i + 1, 4))
            start_rows(i + 1, lax.rem(i + 1, 2), lax.rem(i + 1, 4))

        @pl.when(i + 2 < _NCHB)
        def _():
            start_idx(i + 2, lax.rem(i + 2, 4))

        wait_rows(s2, s4)
        for g in range(_KB // 16):
            ev = lax.iota(jnp.int32, 16) + g * 16
            sl2 = jnp.full((16,), 0, jnp.int32) + s2
            acc = jnp.zeros((16,), jnp.float32)

            def ch(c, acc):
                cc = jnp.full((16,), 0, jnp.int32) + c
                hv = plsc.load_gather(hbuf, [sl2, ev, cc])
                rv = plsc.load_gather(rbuf, [sl2, ev, cc])
                gv = plsc.load_gather(gbuf, [sl2, ev, cc])
                return acc + (hv * rv) * gv

            acc = lax.fori_loop(0, _C, ch, acc, unroll=8)
            rv16 = ridx[s4, pl.ds(g * 16, 16)] * 4
            for c3 in range(3):
                uv = ubuf[s2, c3, pl.ds(g * 16, 16)]
                plsc.addupdate_scatter(facc, [rv16 + c3], acc * uv)
        return 0

    lax.fori_loop(0, _NCHB, body, 0, unroll=False)
    pltpu.sync_copy(facc, out_hbm.at[pl.ds(w * _NPAD * 4, _NPAD * 4)])


@functools.partial(
    pl.kernel,
    out_type=jax.ShapeDtypeStruct((_NW * _NPAD * 4,), jnp.float32),
    mesh=_sc_mesh,
    compiler_params=pltpu.CompilerParams(needs_layout_passes=False),
    scratch_types=[
        pltpu.VMEM((2, _KV), jnp.int32),
        pltpu.VMEM((2, _KV), jnp.int32),
        pltpu.VMEM((2, 4, _KV), jnp.float32),
        pltpu.VMEM((_NPAD * 4,), jnp.float32),
        pltpu.SemaphoreType.DMA((2,)),
        pltpu.SemaphoreType.DMA((2,)),
        pltpu.SemaphoreType.DMA((2,)),
        pltpu.SemaphoreType.DMA,
    ],
)
def _sc_forces_pass(gv_hbm, send_hbm, recv_hbm, z4_hbm, out_hbm,
                    sidx, ridx, gvbuf, facc, sem_s, sem_r, sem_g, sem0):
    cid = lax.axis_index("c")
    sid = lax.axis_index("s")
    w = cid * _NS + sid
    base = w * (_EP // _NW)

    pltpu.async_copy(z4_hbm, facc, sem0).wait()

    def start_chunk(c, slot):
        off = base + c * _KV
        pltpu.async_copy(send_hbm.at[pl.ds(off, _KV)], sidx.at[slot],
                         sem_s.at[slot])
        pltpu.async_copy(recv_hbm.at[pl.ds(off, _KV)], ridx.at[slot],
                         sem_r.at[slot])
        pltpu.async_copy(gv_hbm.at[base // _KV + c], gvbuf.at[slot],
                         sem_g.at[slot])

    def wait_chunk(slot):
        pltpu.make_async_copy(send_hbm.at[pl.ds(0, _KV)], sidx.at[slot],
                              sem_s.at[slot]).wait()
        pltpu.make_async_copy(recv_hbm.at[pl.ds(0, _KV)], ridx.at[slot],
                              sem_r.at[slot]).wait()
        pltpu.make_async_copy(gv_hbm.at[0], gvbuf.at[slot],
                              sem_g.at[slot]).wait()

    start_chunk(0, 0)

    def body(i, _):
        s2 = lax.rem(i, 2)

        @pl.when(i + 1 < _NCHV)
        def _():
            start_chunk(i + 1, lax.rem(i + 1, 2))

        wait_chunk(s2)
        for g in range(_KV // 16):
            ev = lax.iota(jnp.int32, 16) + g * 16
            sl2 = jnp.full((16,), 0, jnp.int32) + s2
            sv = sidx[s2, pl.ds(g * 16, 16)] * 4
            rv = ridx[s2, pl.ds(g * 16, 16)] * 4
            for c3 in range(3):
                gvv = gvbuf[s2, c3, pl.ds(g * 16, 16)]
                plsc.addupdate_scatter(facc, [sv + c3], gvv)
                plsc.addupdate_scatter(facc, [rv + c3], -gvv)
        return 0

    lax.fori_loop(0, _NCHV, body, 0, unroll=False)
    pltpu.sync_copy(facc, out_hbm.at[pl.ds(w * _NPAD * 4, _NPAD * 4)])


_BE = 2048


def _geom(v):
    ln = jnp.sqrt(jnp.sum(v * v, axis=1, keepdims=True) + 1e-12)
    inv = 1.0 / ln
    u = ln / _R_MAX
    Acf = 0.5 * (_P + 1) * (_P + 2)
    Bcf = _P * (_P + 2)
    Ccf = 0.5 * _P * (_P + 1)
    inside = u < 1.0
    fc = jnp.where(inside, 1.0 - Acf * u**_P + Bcf * u**(_P + 1)
                   - Ccf * u**(_P + 2), 0.0)
    dfc = jnp.where(inside, (-Acf * _P * u**(_P - 1) + Bcf * (_P + 1) * u**_P
                             - Ccf * (_P + 2) * u**(_P + 1)) / _R_MAX, 0.0)
    kk = (jax.lax.broadcasted_iota(jnp.int32, (1, _NB), 1) + 1
          ).astype(jnp.float32)
    arg = (kk * jnp.pi / _R_MAX) * ln
    sin_, cos_ = jnp.sin(arg), jnp.cos(arg)
    pref = jnp.float32((2.0 / _R_MAX) ** 0.5)
    bess = pref * sin_ * inv
    ef = bess * fc
    dbess = pref * ((kk * jnp.pi / _R_MAX) * cos_ * inv - sin_ * inv * inv)
    def_dl = dbess * fc + bess * dfc
    return ln, inv, ef, def_dl


def _dot(a, b):
    return jax.lax.dot_general(a, b, (((1,), (0,)), ((), ())),
                               preferred_element_type=jnp.float32,
                               precision=jax.lax.Precision.DEFAULT)


def _edge_fwd_kernel(vec_ref, w_ref_tree, u4_ref, r00_ref, r10_ref,
                     r01_ref, r11_ref):
    v = vec_ref[...]
    ln, inv, ef, _ = _geom(v)
    u4_ref[...] = v * inv
    outs = ((r00_ref, r10_ref), (r01_ref, r11_ref))
    for li in range(2):
        w1, w2, w3 = w_ref_tree[3 * li], w_ref_tree[3 * li + 1], w_ref_tree[3 * li + 2]
        r1 = _silu(_dot(ef, w1[...]))
        r2 = _silu(_dot(r1, w2[...]))
        r3 = _dot(r2, w3[...])
        outs[li][0][...] = r3[:, :_C]
        outs[li][1][...] = r3[:, _C:]


def _tc_edge_fwd(vec4, params):
    wl = []
    for lp in params["layers"]:
        wl += [lp["Wr1"], lp["Wr2"], lp["Wr3"]]
    nb = _EP // _BE
    full = lambda s: pl.BlockSpec(s, lambda i: tuple(0 for _ in s))
    outs = [jax.ShapeDtypeStruct((_EP, 4), jnp.float32)] + [
        jax.ShapeDtypeStruct((_EP, _C), jnp.float32)] * 4
    def kbody(vec_ref, *rest):
        wrefs = rest[:6]
        outr = rest[6:]
        _edge_fwd_kernel(vec_ref, wrefs, *outr)
    return pl.pallas_call(
        kbody,
        grid=(nb,),
        in_specs=[pl.BlockSpec((_BE, 4), lambda i: (i, 0))]
        + [full((_NB, 64)), full((64, 64)), full((64, 2 * _C))] * 2,
        out_specs=[pl.BlockSpec((_BE, 4), lambda i: (i, 0))]
        + [pl.BlockSpec((_BE, _C), lambda i: (i, 0))] * 4,
        out_shape=outs,
    )(vec4, *wl)


def _tc_edge_bwd(vec4, gr0, gr1, params):
    wl = []
    for lp in params["layers"]:
        wl += [lp["Wr1"], lp["Wr2"], lp["Wr3"]]
    nb = _EP // _BE
    full = lambda s: pl.BlockSpec(s, lambda i: tuple(0 for _ in s))

    def kbody(vec_ref, g0_ref, g1_ref, *rest, gv_ref):
        wrefs = rest
        v = vec_ref[...]
        ln, inv, ef, def_dl = _geom(v)
        gl = jnp.zeros((_BE, 1), jnp.float32)
        for li, gref in ((0, g0_ref), (1, g1_ref)):
            w1, w2, w3 = (wrefs[3 * li][...], wrefs[3 * li + 1][...],
                          wrefs[3 * li + 2][...])
            z1 = _dot(ef, w1)
            r1 = _silu(z1)
            z2 = _dot(r1, w2)
            r2 = _silu(z2)
            g_r2 = _dot(gref[...], w3[:, :_C].T)
            g_z2 = g_r2 * _dsilu(z2)
            g_r1 = _dot(g_z2, w2.T)
            g_z1 = g_r1 * _dsilu(z1)
            g_ef = _dot(g_z1, w1.T)
            gl = gl + jnp.sum(g_ef * def_dl, axis=1, keepdims=True)
        gv_ref[...] = gl * (v * inv)

    def kb(*refs):
        return kbody(*refs[:-1], gv_ref=refs[-1])

    return pl.pallas_call(
        kb,
        grid=(nb,),
        in_specs=[pl.BlockSpec((_BE, 4), lambda i: (i, 0)),
                  pl.BlockSpec((_BE, _C), lambda i: (i, 0)),
                  pl.BlockSpec((_BE, _C), lambda i: (i, 0))]
        + [full((_NB, 64)), full((64, 64)), full((64, 2 * _C))] * 2,
        out_specs=pl.BlockSpec((_BE, 4), lambda i: (i, 0)),
        out_shape=jax.ShapeDtypeStruct((_EP, 4), jnp.float32),
    )(vec4, gr0, gr1, *wl)


def _silu(x):
    return x * jax.nn.sigmoid(x)


def _dsilu(x):
    s = jax.nn.sigmoid(x)
    return s * (1 + x * (1 - s))


def _segsum_kernel(batch_ref, vals_ref, out_ref):
    i = pl.program_id(0)

    @pl.when(i == 0)
    def _init():
        out_ref[...] = jnp.zeros_like(out_ref)

    b = batch_ref[...]
    gids = jax.lax.broadcasted_iota(jnp.int32, (1, _GPAD), 1)
    onehot = (b == gids).astype(jnp.float32)
    out_ref[...] += jax.lax.dot_general(
        onehot, vals_ref[...], (((0,), (0,)), ((), ())),
        preferred_element_type=jnp.float32)


def _graph_segment_sums(batch, vals):
    K = vals.shape[1]
    BN = 2048
    nb = _NPAD // BN
    batch_p = jnp.full((_NPAD, 1), _GPAD - 1, jnp.int32).at[:_N, 0].set(batch.astype(jnp.int32))
    vals_p = jnp.zeros((_NPAD, K), jnp.float32).at[:_N].set(vals)
    out = pl.pallas_call(
        _segsum_kernel,
        grid=(nb,),
        in_specs=[
            pl.BlockSpec((BN, 1), lambda i: (i, 0)),
            pl.BlockSpec((BN, K), lambda i: (i, 0)),
        ],
        out_specs=pl.BlockSpec((_GPAD, K), lambda i: (0, 0)),
        out_shape=jax.ShapeDtypeStruct((_GPAD, K), jnp.float32),
    )(batch_p, vals_p)
    return out[:_G]


def kernel(positions, node_attrs, charges, params, edge_index, batch):
    send_p = jnp.zeros((_EP,), jnp.int32).at[:_E].set(
        edge_index[0].astype(jnp.int32))
    recv_p = jnp.zeros((_EP,), jnp.int32).at[:_E].set(
        edge_index[1].astype(jnp.int32))
    zeros4 = jnp.zeros((_NPAD * 4,), jnp.float32)
    vec = positions[recv_p] - positions[send_p]
    vec4 = jnp.zeros((_EP, 4), jnp.float32).at[:, :3].set(vec)
    vec4 = vec4.at[_E:, 0].set(3.0 * _R_MAX)

    u4, R00, R10, R01, R11 = _tc_edge_fwd(vec4, params)
    u4c = jnp.transpose(u4.reshape(_EP // 64, 64, 4), (0, 2, 1))

    node_e0 = node_attrs @ params["atomic_energies"]
    h0 = node_attrs @ params["W_embed"]

    h_in = h0
    saved = []
    he = []
    dparts = []
    for lp, R0, R1 in zip(params["layers"], (R00, R01), (R10, R11)):
        agg0 = _agg0_full(h_in, R0, send_p, recv_p) / _AVG_N
        h_out = h_in @ lp["Wsc"] + _silu(agg0)
        gate = _silu(agg0 @ lp["Wg"])
        gw = gate * lp["w_d"][None, :]
        dparts.append(_sc_dipole_pass(h_in, gw, R1, u4c, send_p, recv_p,
                                      zeros4))
        he.append(h_out @ lp["w_e"])
        saved.append(dict(R0=R0, h_in=h_in, agg0=agg0))
        h_in = h_out

    lp0, lp1 = params["layers"]
    sv0, sv1 = saved
    ga1 = lp1["w_e"][None, :] * _dsilu(sv1["agg0"]) / _AVG_N
    zc = jnp.zeros((_NROWS_SUB, _C), jnp.float32)
    g_R0_1, s1p = _bwd1_pass(sv1["h_in"], ga1, sv1["R0"], send_p, recv_p, zc)
    s1p = s1p.reshape(_NC, _NPAD, _C)
    g_hout0 = (lp0["w_e"][None, :] + (lp1["Wsc"] @ lp1["w_e"])[None, :]
               + s1p[0, :_N] + s1p[1, :_N])
    ga0 = g_hout0 * _dsilu(sv0["agg0"]) / _AVG_N
    g_R0_0 = _bwd0_pass(sv0["h_in"], ga0, send_p, recv_p)

    gv4 = _tc_edge_bwd(vec4, g_R0_0, g_R0_1, params)
    gv4c = jnp.transpose(gv4.reshape(_EP // 128, 128, 4), (0, 2, 1))
    fparts = _sc_forces_pass(gv4c, send_p, recv_p,
                             zeros4).reshape(_NW, _NPAD, 4)
    forces = jnp.sum(fparts, axis=0)[:_N, :3]

    dsum = (dparts[0] + dparts[1]).reshape(_NW, _NPAD, 4)
    atomic_dipoles = jnp.sum(dsum, axis=0)[:_N, :3] / _AVG_N

    vals = jnp.concatenate(
        [node_e0[:, None], he[0][:, None], he[1][:, None], atomic_dipoles,
         charges[:, None] * positions], axis=1)
    segs = _graph_segment_sums(batch, vals)
    e0, e1, e2 = segs[:, 0], segs[:, 1], segs[:, 2]
    total_dipole = segs[:, 3:6] + segs[:, 6:9]
    contributions = jnp.stack([e0, e1, e2], axis=-1)
    total_energy = e0 + e1 + e2
    return (total_energy, node_e0, contributions, forces, total_dipole, atomic_dipoles)
